# baseline (speedup 1.0000x reference)
import jax
import jax.numpy as jnp
from jax.experimental import pallas as pl
from jax.experimental.pallas import tpu as pltpu

_MAX_ROWS = 32768
_NBUF = 3
_LOOKAHEAD = 2

_SIZES = [4096, 4096, 24576] + [32768] * 6 + [16384, 8192, 4096, 4096]


def _chunks(per_tensor_rows):
    assert sum(_SIZES) == per_tensor_rows
    seq = []
    for t in range(2):
        off = 0
        for s in _SIZES:
            seq.append((t, off, s))
            off += s
    return seq


def _copy_body(k_ref, v_ref, ok_ref, ov_ref, *scratch):
    bufs = scratch[:_NBUF]
    gsem = scratch[_NBUF]
    ssem = scratch[_NBUF + 1]
    seq = _chunks(k_ref.shape[0])
    n = len(seq)

    def src(c):
        t, off, s = seq[c]
        return (k_ref, v_ref)[t].at[pl.ds(off, s)]

    def dst(c):
        t, off, s = seq[c]
        return (ok_ref, ov_ref)[t].at[pl.ds(off, s)]

    def buf(c):
        s = seq[c][2]
        b = bufs[c % _NBUF]
        if s == _MAX_ROWS:
            return b
        return b.at[pl.ds(0, s)]

    ins = [None] * n
    outs = [None] * n
    out_waited = [False] * n

    for c in range(min(_LOOKAHEAD, n)):
        ins[c] = pltpu.make_async_copy(src(c), buf(c), gsem.at[c % _NBUF])
        ins[c].start()
    for c in range(n):
        ins[c].wait()
        outs[c] = pltpu.make_async_copy(buf(c), dst(c), ssem.at[c % _NBUF])
        outs[c].start()
        nxt = c + _LOOKAHEAD
        if nxt < n:
            if nxt >= _NBUF:
                outs[nxt - _NBUF].wait()
                out_waited[nxt - _NBUF] = True
            ins[nxt] = pltpu.make_async_copy(src(nxt), buf(nxt), gsem.at[nxt % _NBUF])
            ins[nxt].start()
    for c in range(n):
        if not out_waited[c]:
            outs[c].wait()


def kernel(new_k, new_v):
    shape = new_k.shape
    rows = new_k.size // 128
    k2 = new_k.reshape(rows, 128)
    v2 = new_v.reshape(rows, 128)
    out2 = pl.pallas_call(
        _copy_body,
        in_specs=[
            pl.BlockSpec(memory_space=pl.ANY),
            pl.BlockSpec(memory_space=pl.ANY),
        ],
        out_specs=[
            pl.BlockSpec(memory_space=pl.ANY),
            pl.BlockSpec(memory_space=pl.ANY),
        ],
        out_shape=(
            jax.ShapeDtypeStruct((rows, 128), jnp.float32),
            jax.ShapeDtypeStruct((rows, 128), jnp.float32),
        ),
        scratch_shapes=(
            [pltpu.VMEM((_MAX_ROWS, 128), jnp.float32)] * _NBUF
            + [pltpu.SemaphoreType.DMA((_NBUF,)), pltpu.SemaphoreType.DMA((_NBUF,))]
        ),
    )(k2, v2)
    return (out2[0].reshape(shape), out2[1].reshape(shape))


# speedup vs baseline: 1.0126x; 1.0126x over previous
import jax
import jax.numpy as jnp
from jax.experimental import pallas as pl
from jax.experimental.pallas import tpu as pltpu

_MAX_ROWS = 32768
_NBUF = 3
_LOOKAHEAD = 2

_SIZES = [8192] + [32768] * 7 + [16384] + [8192]


def _chunks(per_tensor_rows):
    assert sum(_SIZES) == per_tensor_rows
    seq = []
    for t in range(2):
        off = 0
        for s in _SIZES:
            seq.append((t, off, s))
            off += s
    return seq


def _copy_body(k_ref, v_ref, ok_ref, ov_ref, *scratch):
    bufs = scratch[:_NBUF]
    gsem = scratch[_NBUF]
    ssem = scratch[_NBUF + 1]
    seq = _chunks(k_ref.shape[0])
    n = len(seq)

    def src(c):
        t, off, s = seq[c]
        return (k_ref, v_ref)[t].at[pl.ds(off, s)]

    def dst(c):
        t, off, s = seq[c]
        return (ok_ref, ov_ref)[t].at[pl.ds(off, s)]

    def buf(c):
        s = seq[c][2]
        b = bufs[c % _NBUF]
        if s == _MAX_ROWS:
            return b
        return b.at[pl.ds(0, s)]

    ins = [None] * n
    outs = [None] * n
    out_waited = [False] * n

    for c in range(min(_LOOKAHEAD, n)):
        ins[c] = pltpu.make_async_copy(src(c), buf(c), gsem.at[c % _NBUF])
        ins[c].start()
    for c in range(n):
        ins[c].wait()
        outs[c] = pltpu.make_async_copy(buf(c), dst(c), ssem.at[c % _NBUF])
        outs[c].start()
        nxt = c + _LOOKAHEAD
        if nxt < n:
            if nxt >= _NBUF:
                outs[nxt - _NBUF].wait()
                out_waited[nxt - _NBUF] = True
            ins[nxt] = pltpu.make_async_copy(src(nxt), buf(nxt), gsem.at[nxt % _NBUF])
            ins[nxt].start()
    for c in range(n):
        if not out_waited[c]:
            outs[c].wait()


def kernel(new_k, new_v):
    shape = new_k.shape
    rows = new_k.size // 128
    k2 = new_k.reshape(rows, 128)
    v2 = new_v.reshape(rows, 128)
    out2 = pl.pallas_call(
        _copy_body,
        in_specs=[
            pl.BlockSpec(memory_space=pl.ANY),
            pl.BlockSpec(memory_space=pl.ANY),
        ],
        out_specs=[
            pl.BlockSpec(memory_space=pl.ANY),
            pl.BlockSpec(memory_space=pl.ANY),
        ],
        out_shape=(
            jax.ShapeDtypeStruct((rows, 128), jnp.float32),
            jax.ShapeDtypeStruct((rows, 128), jnp.float32),
        ),
        scratch_shapes=(
            [pltpu.VMEM((_MAX_ROWS, 128), jnp.float32)] * _NBUF
            + [pltpu.SemaphoreType.DMA((_NBUF,)), pltpu.SemaphoreType.DMA((_NBUF,))]
        ),
    )(k2, v2)
    return (out2[0].reshape(shape), out2[1].reshape(shape))
